# initial kernel scaffold (unmeasured)
import jax
import jax.numpy as jnp
from jax import lax
from jax.experimental import pallas as pl
from jax.experimental.pallas import tpu as pltpu

N_DEV = 4
SQ = 1024
SKV = 1024
HQ = 8
DH = 128
D = 1024
SCALE = 0.08838834764831843
WINDOW = 128
NGLOB = 32

OVERLAP = False


def kernel(x, Wq, K_ext, V_ext, Wo):
    x2 = x.reshape(SQ, D)
    Kt = K_ext.reshape(SKV, HQ, DH).transpose(1, 0, 2)
    Vt = V_ext.reshape(SKV, HQ, DH).transpose(1, 0, 2)

    def body(x_ref, wq_ref, k_ref, v_ref, wo_ref, out_ref,
             kbuf, vbuf, qs, ksend, krecv, vsend, vrecv):
        my = lax.axis_index("i")
        left = lax.rem(my + N_DEV - 1, N_DEV)
        right = lax.rem(my + 1, N_DEV)

        barrier = pltpu.get_barrier_semaphore()
        for nbr in (left, right):
            pl.semaphore_signal(barrier, inc=1, device_id=(nbr,),
                                device_id_type=pl.DeviceIdType.MESH)
        pl.semaphore_wait(barrier, 2)

        kbuf[0, :, :, :] = k_ref[:, :, :]
        vbuf[0, :, :, :] = v_ref[:, :, :]

        def make(buf, ssem, rsem, h):
            return pltpu.make_async_remote_copy(
                src_ref=buf.at[h],
                dst_ref=buf.at[h + 1],
                send_sem=ssem.at[h],
                recv_sem=rsem.at[h],
                device_id=(right,),
                device_id_type=pl.DeviceIdType.MESH,
            )

        state = [None] * HQ
        sends = []
        for s in range(N_DEV):
            if s < N_DEV - 1:
                kr = make(kbuf, ksend, krecv, s)
                vr = make(vbuf, vsend, vrecv, s)
                kr.start()
                vr.start()
                sends.append((kr, vr))
                if not OVERLAP:
                    kr.wait()
                    vr.wait()
            if s == 0:
                qs[:, :] = jnp.dot(x_ref[:, :], wq_ref[:, :],
                                   preferred_element_type=jnp.float32)

            origin = lax.rem(my + N_DEV - s, N_DEV)
            row = my * SQ + lax.broadcasted_iota(jnp.int32, (SQ, SKV), 0)
            col = origin * SKV + lax.broadcasted_iota(jnp.int32, (SQ, SKV), 1)
            mask = ((jnp.abs(row - col) <= WINDOW)
                    | (col < NGLOB) | (row < NGLOB))
            for h in range(HQ):
                qh = qs[:, h * DH:(h + 1) * DH]
                kh = kbuf[s, h]
                sc = lax.dot_general(
                    qh, kh, (((1,), (1,)), ((), ())),
                    preferred_element_type=jnp.float32) * SCALE
                sc = jnp.where(mask, sc, jnp.float32(-1e9))
                mx = jnp.max(sc, axis=1, keepdims=True)
                if state[h] is None:
                    m = mx
                    p = jnp.exp(sc - m)
                    l = jnp.sum(p, axis=1, keepdims=True)
                    acc = jnp.dot(p, vbuf[s, h],
                                  preferred_element_type=jnp.float32)
                else:
                    m0, l0, a0 = state[h]
                    m = jnp.maximum(m0, mx)
                    corr = jnp.exp(m0 - m)
                    p = jnp.exp(sc - m)
                    l = l0 * corr + jnp.sum(p, axis=1, keepdims=True)
                    acc = a0 * corr + jnp.dot(
                        p, vbuf[s, h], preferred_element_type=jnp.float32)
                state[h] = (m, l, acc)

            if OVERLAP and s < N_DEV - 1:
                kr.wait_recv()
                vr.wait_recv()

        if OVERLAP:
            for kr, vr in sends:
                kr.wait_send()
                vr.wait_send()

        out = jnp.zeros((SQ, D), jnp.float32)
        for h in range(HQ):
            m, l, acc = state[h]
            out = out + jnp.dot(acc / l, wo_ref[h * DH:(h + 1) * DH, :],
                                preferred_element_type=jnp.float32)
        out_ref[:, :] = out

    out2 = pl.pallas_call(
        body,
        out_shape=jax.ShapeDtypeStruct((SQ, D), jnp.float32),
        in_specs=[pl.BlockSpec(memory_space=pltpu.VMEM)] * 5,
        out_specs=pl.BlockSpec(memory_space=pltpu.VMEM),
        scratch_shapes=[
            pltpu.VMEM((N_DEV, HQ, SKV, DH), jnp.float32),
            pltpu.VMEM((N_DEV, HQ, SKV, DH), jnp.float32),
            pltpu.VMEM((SQ, D), jnp.float32),
            pltpu.SemaphoreType.DMA((N_DEV - 1,)),
            pltpu.SemaphoreType.DMA((N_DEV - 1,)),
            pltpu.SemaphoreType.DMA((N_DEV - 1,)),
            pltpu.SemaphoreType.DMA((N_DEV - 1,)),
        ],
        compiler_params=pltpu.CompilerParams(collective_id=0),
    )(x2, Wq, Kt, Vt, Wo)
    return out2.reshape(1, SQ, D)


# baseline (device time: 283781 ns/iter reference)
import jax
import jax.numpy as jnp
from jax import lax
from jax.experimental import pallas as pl
from jax.experimental.pallas import tpu as pltpu

N_DEV = 4
SQ = 1024
SKV = 1024
HQ = 8
DH = 128
D = 1024
SCALE = 0.08838834764831843
WINDOW = 128
NGLOB = 32
QB = 256

OVERLAP = False


def kernel(x, Wq, K_ext, V_ext, Wo):
    x2 = x.reshape(SQ, D).astype(jnp.bfloat16)
    Wq2 = Wq.astype(jnp.bfloat16)
    Wo2 = Wo.astype(jnp.bfloat16)
    Kt = K_ext.reshape(SKV, HQ, DH).transpose(1, 0, 2).astype(jnp.bfloat16)
    Vt = V_ext.reshape(SKV, HQ, DH).transpose(1, 0, 2).astype(jnp.bfloat16)

    def body(x_ref, wq_ref, k_ref, v_ref, wo_ref, out_ref,
             kbuf, vbuf, qs, m_ref, l_ref, acc_ref,
             ksend, krecv, vsend, vrecv):
        my = lax.axis_index("i")
        left = lax.rem(my + N_DEV - 1, N_DEV)
        right = lax.rem(my + 1, N_DEV)

        barrier = pltpu.get_barrier_semaphore()
        for nbr in (left, right):
            pl.semaphore_signal(barrier, inc=1, device_id=(nbr,),
                                device_id_type=pl.DeviceIdType.MESH)
        pl.semaphore_wait(barrier, 2)

        kbuf[0:HQ, :, :] = k_ref[:, :, :]
        vbuf[0:HQ, :, :] = v_ref[:, :, :]

        def make(buf, ssem, rsem, s):
            return pltpu.make_async_remote_copy(
                src_ref=buf.at[pl.ds(s * HQ, HQ)],
                dst_ref=buf.at[pl.ds((s + 1) * HQ, HQ)],
                send_sem=ssem.at[s],
                recv_sem=rsem.at[s],
                device_id=(right,),
                device_id_type=pl.DeviceIdType.MESH,
            )

        sends = []
        for s in range(N_DEV):
            if s < N_DEV - 1:
                kr = make(kbuf, ksend, krecv, s)
                vr = make(vbuf, vsend, vrecv, s)
                kr.start()
                vr.start()
                sends.append((kr, vr))
                if not OVERLAP:
                    kr.wait()
                    vr.wait()
            if s == 0:
                qs[:, :] = jnp.dot(
                    x_ref[:, :], wq_ref[:, :],
                    preferred_element_type=jnp.float32).astype(jnp.bfloat16)

            origin = lax.rem(my + N_DEV - s, N_DEV)

            def qblock_step(qb, _, s=s):
                q0 = qb * QB
                row = (my * SQ + q0
                       + lax.broadcasted_iota(jnp.int32, (QB, SKV), 0))
                col = (origin * SKV
                       + lax.broadcasted_iota(jnp.int32, (QB, SKV), 1))
                mask = ((jnp.abs(row - col) <= WINDOW)
                        | (col < NGLOB) | (row < NGLOB))
                bias = jnp.where(mask, jnp.float32(0.0), jnp.float32(-1e9))

                def head_step(h, _, s=s):
                    qh = qs[pl.ds(q0, QB), pl.ds(h * DH, DH)]
                    kh = kbuf[pl.ds(s * HQ + h, 1)].reshape(SKV, DH)
                    vh = vbuf[pl.ds(s * HQ + h, 1)].reshape(SKV, DH)
                    sc = lax.dot_general(
                        qh, kh, (((1,), (1,)), ((), ())),
                        preferred_element_type=jnp.float32) * SCALE + bias
                    mx = jnp.max(sc, axis=1, keepdims=True)
                    if s == 0:
                        m = mx
                        p = jnp.exp(sc - m)
                        l = jnp.sum(p, axis=1, keepdims=True)
                        acc = jnp.dot(p.astype(jnp.bfloat16), vh,
                                      preferred_element_type=jnp.float32)
                    else:
                        m0 = m_ref[pl.ds(h, 1), pl.ds(q0, QB)].reshape(QB, 1)
                        m = jnp.maximum(m0, mx)
                        corr = jnp.exp(m0 - m)
                        p = jnp.exp(sc - m)
                        l0 = l_ref[pl.ds(h, 1), pl.ds(q0, QB)].reshape(QB, 1)
                        l = l0 * corr + jnp.sum(p, axis=1, keepdims=True)
                        a0 = acc_ref[pl.ds(h, 1),
                                     pl.ds(q0, QB)].reshape(QB, DH)
                        acc = a0 * corr + jnp.dot(
                            p.astype(jnp.bfloat16), vh,
                            preferred_element_type=jnp.float32)
                    m_ref[pl.ds(h, 1), pl.ds(q0, QB)] = m.reshape(1, QB, 1)
                    l_ref[pl.ds(h, 1), pl.ds(q0, QB)] = l.reshape(1, QB, 1)
                    acc_ref[pl.ds(h, 1), pl.ds(q0, QB)] = (
                        acc.reshape(1, QB, DH))
                    return 0

                lax.fori_loop(0, HQ, head_step, 0)
                return 0

            lax.fori_loop(0, SQ // QB, qblock_step, 0)

            if OVERLAP and s < N_DEV - 1:
                kr.wait_recv()
                vr.wait_recv()

        if OVERLAP:
            for kr, vr in sends:
                kr.wait_send()
                vr.wait_send()

        def ctx_step(h, _):
            acc = acc_ref[pl.ds(h, 1)].reshape(SQ, DH)
            l = l_ref[pl.ds(h, 1)].reshape(SQ, 1)
            qs[:, pl.ds(h * DH, DH)] = (acc / l).astype(jnp.bfloat16)
            return 0

        lax.fori_loop(0, HQ, ctx_step, 0)
        out_ref[:, :] = jnp.dot(qs[:, :], wo_ref[:, :],
                                preferred_element_type=jnp.float32)

    out2 = pl.pallas_call(
        body,
        out_shape=jax.ShapeDtypeStruct((SQ, D), jnp.float32),
        in_specs=[pl.BlockSpec(memory_space=pltpu.VMEM)] * 5,
        out_specs=pl.BlockSpec(memory_space=pltpu.VMEM),
        scratch_shapes=[
            pltpu.VMEM((N_DEV * HQ, SKV, DH), jnp.bfloat16),
            pltpu.VMEM((N_DEV * HQ, SKV, DH), jnp.bfloat16),
            pltpu.VMEM((SQ, D), jnp.bfloat16),
            pltpu.VMEM((HQ, SQ, 1), jnp.float32),
            pltpu.VMEM((HQ, SQ, 1), jnp.float32),
            pltpu.VMEM((HQ, SQ, DH), jnp.float32),
            pltpu.SemaphoreType.DMA((N_DEV - 1,)),
            pltpu.SemaphoreType.DMA((N_DEV - 1,)),
            pltpu.SemaphoreType.DMA((N_DEV - 1,)),
            pltpu.SemaphoreType.DMA((N_DEV - 1,)),
        ],
        compiler_params=pltpu.CompilerParams(
            collective_id=0, vmem_limit_bytes=44 * 1024 * 1024),
    )(x2, Wq2, Kt, Vt, Wo2)
    return out2.reshape(1, SQ, D)


# device time: 196056 ns/iter; 1.4474x vs baseline; 1.4474x over previous
import jax
import jax.numpy as jnp
from jax import lax
from jax.experimental import pallas as pl
from jax.experimental.pallas import tpu as pltpu

N_DEV = 4
SQ = 1024
SKV = 1024
HQ = 8
DH = 128
D = 1024
SCALE = 0.08838834764831843
WINDOW = 128
NGLOB = 32
QB = 256

OVERLAP = True


def kernel(x, Wq, K_ext, V_ext, Wo):
    x2 = x.reshape(SQ, D).astype(jnp.bfloat16)
    Wq2 = Wq.astype(jnp.bfloat16)
    Wo2 = Wo.astype(jnp.bfloat16)
    Kt = K_ext.reshape(SKV, HQ, DH).transpose(1, 0, 2).astype(jnp.bfloat16)
    Vt = V_ext.reshape(SKV, HQ, DH).transpose(1, 0, 2).astype(jnp.bfloat16)

    def body(x_ref, wq_ref, k_ref, v_ref, wo_ref, out_ref,
             kbuf, vbuf, qs, m_ref, l_ref, acc_ref,
             ksend, krecv, vsend, vrecv):
        my = lax.axis_index("i")
        left = lax.rem(my + N_DEV - 1, N_DEV)
        right = lax.rem(my + 1, N_DEV)

        barrier = pltpu.get_barrier_semaphore()
        for nbr in (left, right):
            pl.semaphore_signal(barrier, inc=1, device_id=(nbr,),
                                device_id_type=pl.DeviceIdType.MESH)
        pl.semaphore_wait(barrier, 2)

        kbuf[0:HQ, :, :] = k_ref[:, :, :]
        vbuf[0:HQ, :, :] = v_ref[:, :, :]

        def make(buf, ssem, rsem, s):
            return pltpu.make_async_remote_copy(
                src_ref=buf.at[pl.ds(s * HQ, HQ)],
                dst_ref=buf.at[pl.ds((s + 1) * HQ, HQ)],
                send_sem=ssem.at[s],
                recv_sem=rsem.at[s],
                device_id=(right,),
                device_id_type=pl.DeviceIdType.MESH,
            )

        sends = []
        for s in range(N_DEV):
            if s < N_DEV - 1:
                kr = make(kbuf, ksend, krecv, s)
                vr = make(vbuf, vsend, vrecv, s)
                kr.start()
                vr.start()
                sends.append((kr, vr))
                if not OVERLAP:
                    kr.wait()
                    vr.wait()
            if s == 0:
                qs[:, :] = jnp.dot(
                    x_ref[:, :], wq_ref[:, :],
                    preferred_element_type=jnp.float32).astype(jnp.bfloat16)

            origin = lax.rem(my + N_DEV - s, N_DEV)

            def qblock_step(qb, _, s=s):
                q0 = qb * QB
                row = (my * SQ + q0
                       + lax.broadcasted_iota(jnp.int32, (QB, SKV), 0))
                col = (origin * SKV
                       + lax.broadcasted_iota(jnp.int32, (QB, SKV), 1))
                mask = ((jnp.abs(row - col) <= WINDOW)
                        | (col < NGLOB) | (row < NGLOB))
                bias = jnp.where(mask, jnp.float32(0.0), jnp.float32(-1e9))

                def head_step(h, _, s=s):
                    qh = qs[pl.ds(q0, QB), pl.ds(h * DH, DH)]
                    kh = kbuf[pl.ds(s * HQ + h, 1)].reshape(SKV, DH)
                    vh = vbuf[pl.ds(s * HQ + h, 1)].reshape(SKV, DH)
                    sc = lax.dot_general(
                        qh, kh, (((1,), (1,)), ((), ())),
                        preferred_element_type=jnp.float32) * SCALE + bias
                    mx = jnp.max(sc, axis=1, keepdims=True)
                    if s == 0:
                        m = mx
                        p = jnp.exp(sc - m)
                        l = jnp.sum(p, axis=1, keepdims=True)
                        acc = jnp.dot(p.astype(jnp.bfloat16), vh,
                                      preferred_element_type=jnp.float32)
                    else:
                        m0 = m_ref[pl.ds(h, 1), pl.ds(q0, QB)].reshape(QB, 1)
                        m = jnp.maximum(m0, mx)
                        corr = jnp.exp(m0 - m)
                        p = jnp.exp(sc - m)
                        l0 = l_ref[pl.ds(h, 1), pl.ds(q0, QB)].reshape(QB, 1)
                        l = l0 * corr + jnp.sum(p, axis=1, keepdims=True)
                        a0 = acc_ref[pl.ds(h, 1),
                                     pl.ds(q0, QB)].reshape(QB, DH)
                        acc = a0 * corr + jnp.dot(
                            p.astype(jnp.bfloat16), vh,
                            preferred_element_type=jnp.float32)
                    m_ref[pl.ds(h, 1), pl.ds(q0, QB)] = m.reshape(1, QB, 1)
                    l_ref[pl.ds(h, 1), pl.ds(q0, QB)] = l.reshape(1, QB, 1)
                    acc_ref[pl.ds(h, 1), pl.ds(q0, QB)] = (
                        acc.reshape(1, QB, DH))
                    return 0

                lax.fori_loop(0, HQ, head_step, 0)
                return 0

            lax.fori_loop(0, SQ // QB, qblock_step, 0)

            if OVERLAP and s < N_DEV - 1:
                kr.wait_recv()
                vr.wait_recv()

        if OVERLAP:
            for kr, vr in sends:
                kr.wait_send()
                vr.wait_send()

        def ctx_step(h, _):
            acc = acc_ref[pl.ds(h, 1)].reshape(SQ, DH)
            l = l_ref[pl.ds(h, 1)].reshape(SQ, 1)
            qs[:, pl.ds(h * DH, DH)] = (acc / l).astype(jnp.bfloat16)
            return 0

        lax.fori_loop(0, HQ, ctx_step, 0)
        out_ref[:, :] = jnp.dot(qs[:, :], wo_ref[:, :],
                                preferred_element_type=jnp.float32)

    out2 = pl.pallas_call(
        body,
        out_shape=jax.ShapeDtypeStruct((SQ, D), jnp.float32),
        in_specs=[pl.BlockSpec(memory_space=pltpu.VMEM)] * 5,
        out_specs=pl.BlockSpec(memory_space=pltpu.VMEM),
        scratch_shapes=[
            pltpu.VMEM((N_DEV * HQ, SKV, DH), jnp.bfloat16),
            pltpu.VMEM((N_DEV * HQ, SKV, DH), jnp.bfloat16),
            pltpu.VMEM((SQ, D), jnp.bfloat16),
            pltpu.VMEM((HQ, SQ, 1), jnp.float32),
            pltpu.VMEM((HQ, SQ, 1), jnp.float32),
            pltpu.VMEM((HQ, SQ, DH), jnp.float32),
            pltpu.SemaphoreType.DMA((N_DEV - 1,)),
            pltpu.SemaphoreType.DMA((N_DEV - 1,)),
            pltpu.SemaphoreType.DMA((N_DEV - 1,)),
            pltpu.SemaphoreType.DMA((N_DEV - 1,)),
        ],
        compiler_params=pltpu.CompilerParams(
            collective_id=0, vmem_limit_bytes=44 * 1024 * 1024),
    )(x2, Wq2, Kt, Vt, Wo2)
    return out2.reshape(1, SQ, D)


# device time: 181680 ns/iter; 1.5620x vs baseline; 1.0791x over previous
import jax
import jax.numpy as jnp
from jax import lax
from jax.experimental import pallas as pl
from jax.experimental.pallas import tpu as pltpu

N_DEV = 4
SQ = 1024
SKV = 1024
HQ = 8
DH = 128
D = 1024
SCALE = 0.08838834764831843
WINDOW = 128
NGLOB = 32
QB = 256
KB = 256

OVERLAP = True


def kernel(x, Wq, K_ext, V_ext, Wo):
    x2 = x.reshape(SQ, D).astype(jnp.bfloat16)
    Wq2 = Wq.astype(jnp.bfloat16)
    Wo2 = Wo.astype(jnp.bfloat16)
    Kt = K_ext.reshape(SKV, HQ, DH).transpose(1, 0, 2).astype(jnp.bfloat16)
    Vt = V_ext.reshape(SKV, HQ, DH).transpose(1, 0, 2).astype(jnp.bfloat16)

    def body(x_ref, wq_ref, k_ref, v_ref, wo_ref, out_ref,
             kbuf, vbuf, qs, l_ref, acc_ref,
             ksend, krecv, vsend, vrecv):
        my = lax.axis_index("i")
        left = lax.rem(my + N_DEV - 1, N_DEV)
        right = lax.rem(my + 1, N_DEV)

        barrier = pltpu.get_barrier_semaphore()
        for nbr in (left, right):
            pl.semaphore_signal(barrier, inc=1, device_id=(nbr,),
                                device_id_type=pl.DeviceIdType.MESH)
        pl.semaphore_wait(barrier, 2)

        kbuf[0:HQ, :, :] = k_ref[:, :, :]
        vbuf[0:HQ, :, :] = v_ref[:, :, :]

        l_ref[:, :, :] = jnp.zeros((HQ, SQ, 1), jnp.float32)
        acc_ref[:, :, :] = jnp.zeros((HQ, SQ, DH), jnp.float32)

        def make(buf, ssem, rsem, s):
            return pltpu.make_async_remote_copy(
                src_ref=buf.at[pl.ds(s * HQ, HQ)],
                dst_ref=buf.at[pl.ds((s + 1) * HQ, HQ)],
                send_sem=ssem.at[s],
                recv_sem=rsem.at[s],
                device_id=(right,),
                device_id_type=pl.DeviceIdType.MESH,
            )

        sends = []
        for s in range(N_DEV):
            if s < N_DEV - 1:
                kr = make(kbuf, ksend, krecv, s)
                vr = make(vbuf, vsend, vrecv, s)
                kr.start()
                vr.start()
                sends.append((kr, vr))
                if not OVERLAP:
                    kr.wait()
                    vr.wait()
            if s == 0:
                qs[:, :] = jnp.dot(
                    x_ref[:, :], wq_ref[:, :],
                    preferred_element_type=jnp.float32).astype(jnp.bfloat16)

            origin = lax.rem(my + N_DEV - s, N_DEV)

            def qblock_step(qb, _, s=s):
                q0 = qb * QB
                qlo = my * SQ + q0

                def kvtile_step(kb, _, s=s):
                    k0 = kb * KB
                    klo = origin * SKV + k0
                    band = ((klo <= qlo + (QB - 1) + WINDOW)
                            & (klo + (KB - 1) >= qlo - WINDOW))
                    active = band | (klo < NGLOB) | (qlo < NGLOB)

                    @pl.when(active)
                    def _():
                        row = qlo + lax.broadcasted_iota(
                            jnp.int32, (QB, KB), 0)
                        col = klo + lax.broadcasted_iota(
                            jnp.int32, (QB, KB), 1)
                        mask = ((jnp.abs(row - col) <= WINDOW)
                                | (col < NGLOB) | (row < NGLOB))
                        bias = jnp.where(mask, jnp.float32(0.0),
                                         jnp.float32(-1e9))

                        def head_step(h, _, s=s):
                            qh = qs[pl.ds(q0, QB), pl.ds(h * DH, DH)]
                            kh = kbuf[pl.ds(s * HQ + h, 1),
                                      pl.ds(k0, KB), :].reshape(KB, DH)
                            vh = vbuf[pl.ds(s * HQ + h, 1),
                                      pl.ds(k0, KB), :].reshape(KB, DH)
                            sc = lax.dot_general(
                                qh, kh, (((1,), (1,)), ((), ())),
                                preferred_element_type=jnp.float32
                            ) * SCALE + bias
                            p = jnp.exp(sc)
                            l0 = l_ref[pl.ds(h, 1),
                                       pl.ds(q0, QB)].reshape(QB, 1)
                            l_ref[pl.ds(h, 1), pl.ds(q0, QB)] = (
                                l0 + jnp.sum(p, axis=1, keepdims=True)
                            ).reshape(1, QB, 1)
                            a0 = acc_ref[pl.ds(h, 1),
                                         pl.ds(q0, QB)].reshape(QB, DH)
                            acc_ref[pl.ds(h, 1), pl.ds(q0, QB)] = (
                                a0 + jnp.dot(
                                    p.astype(jnp.bfloat16), vh,
                                    preferred_element_type=jnp.float32)
                            ).reshape(1, QB, DH)
                            return 0

                        lax.fori_loop(0, HQ, head_step, 0)

                    return 0

                lax.fori_loop(0, SKV // KB, kvtile_step, 0)
                return 0

            lax.fori_loop(0, SQ // QB, qblock_step, 0)

            if OVERLAP and s < N_DEV - 1:
                kr.wait_recv()
                vr.wait_recv()

        if OVERLAP:
            for kr, vr in sends:
                kr.wait_send()
                vr.wait_send()

        def ctx_step(h, _):
            acc = acc_ref[pl.ds(h, 1)].reshape(SQ, DH)
            l = l_ref[pl.ds(h, 1)].reshape(SQ, 1)
            qs[:, pl.ds(h * DH, DH)] = (acc / l).astype(jnp.bfloat16)
            return 0

        lax.fori_loop(0, HQ, ctx_step, 0)
        out_ref[:, :] = jnp.dot(qs[:, :], wo_ref[:, :],
                                preferred_element_type=jnp.float32)

    out2 = pl.pallas_call(
        body,
        out_shape=jax.ShapeDtypeStruct((SQ, D), jnp.float32),
        in_specs=[pl.BlockSpec(memory_space=pltpu.VMEM)] * 5,
        out_specs=pl.BlockSpec(memory_space=pltpu.VMEM),
        scratch_shapes=[
            pltpu.VMEM((N_DEV * HQ, SKV, DH), jnp.bfloat16),
            pltpu.VMEM((N_DEV * HQ, SKV, DH), jnp.bfloat16),
            pltpu.VMEM((SQ, D), jnp.bfloat16),
            pltpu.VMEM((HQ, SQ, 1), jnp.float32),
            pltpu.VMEM((HQ, SQ, DH), jnp.float32),
            pltpu.SemaphoreType.DMA((N_DEV - 1,)),
            pltpu.SemaphoreType.DMA((N_DEV - 1,)),
            pltpu.SemaphoreType.DMA((N_DEV - 1,)),
            pltpu.SemaphoreType.DMA((N_DEV - 1,)),
        ],
        compiler_params=pltpu.CompilerParams(
            collective_id=0, vmem_limit_bytes=44 * 1024 * 1024),
    )(x2, Wq2, Kt, Vt, Wo2)
    return out2.reshape(1, SQ, D)


# device time: 131359 ns/iter; 2.1603x vs baseline; 1.3831x over previous
import jax
import jax.numpy as jnp
from jax import lax
from jax.experimental import pallas as pl
from jax.experimental.pallas import tpu as pltpu

N_DEV = 4
SQ = 1024
SKV = 1024
HQ = 8
DH = 128
D = 1024
SCALE = 0.08838834764831843
WINDOW = 128
NGLOB = 32
QB = 256
KB = 256

OVERLAP = True


def kernel(x, Wq, K_ext, V_ext, Wo):
    x2 = x.reshape(SQ, D).astype(jnp.bfloat16)
    Wq2 = Wq.astype(jnp.bfloat16)
    Wo2 = Wo.astype(jnp.bfloat16)
    Kt = K_ext.reshape(SKV, HQ, DH).transpose(1, 0, 2).astype(jnp.bfloat16)
    Vt = V_ext.reshape(SKV, HQ, DH).transpose(1, 0, 2).astype(jnp.bfloat16)

    def body(x_ref, wq_ref, k_ref, v_ref, wo_ref, out_ref,
             kbuf, vbuf, qs, l_ref, acc_ref,
             ksend, krecv, vsend, vrecv):
        my = lax.axis_index("i")
        even = lax.rem(my, 2) == 0

        barrier = pltpu.get_barrier_semaphore()
        for d in (1, 2, 3):
            pl.semaphore_signal(barrier, inc=1,
                                device_id=(lax.rem(my + d, N_DEV),),
                                device_id_type=pl.DeviceIdType.MESH)
        pl.semaphore_wait(barrier, 3)

        l_ref[:, :, :] = jnp.zeros((HQ, SQ, 1), jnp.float32)
        acc_ref[:, :, :] = jnp.zeros((HQ, SQ, DH), jnp.float32)

        def mk(src, buf, d, ssem, rsem):
            return pltpu.make_async_remote_copy(
                src_ref=src,
                dst_ref=buf.at[pl.ds((d - 1) * HQ, HQ)],
                send_sem=ssem.at[d - 1],
                recv_sem=rsem.at[d - 1],
                device_id=(lax.rem(my + d, N_DEV),),
                device_id_type=pl.DeviceIdType.MESH,
            )

        k1 = mk(k_ref, kbuf, 1, ksend, krecv)
        v1 = mk(v_ref, vbuf, 1, vsend, vrecv)
        k3 = mk(k_ref, kbuf, 3, ksend, krecv)
        v3 = mk(v_ref, vbuf, 3, vsend, vrecv)
        k1.start()
        v1.start()
        k3.start()
        v3.start()

        @pl.when(even)
        def _():
            k2 = mk(k_ref, kbuf, 2, ksend, krecv)
            v2 = mk(v_ref, vbuf, 2, vsend, vrecv)
            k2.start()
            v2.start()

        qs[:, :] = jnp.dot(
            x_ref[:, :], wq_ref[:, :],
            preferred_element_type=jnp.float32).astype(jnp.bfloat16)

        for s, before in ((0, None), (1, (k1, v1)), (3, (k3, v3)), (2, None)):
            if before is not None:
                before[0].wait_recv()
                before[1].wait_recv()
            if s == 2:
                @pl.when(even)
                def _():
                    k2r = mk(k_ref, kbuf, 2, ksend, krecv)
                    v2r = mk(v_ref, vbuf, 2, vsend, vrecv)
                    k2r.wait_recv()
                    v2r.wait_recv()

            origin = lax.rem(my + N_DEV - s, N_DEV)

            def qblock_step(qb, _, s=s):
                q0 = qb * QB
                qlo = my * SQ + q0

                def kvtile_step(kb, _, s=s):
                    k0 = kb * KB
                    klo = origin * SKV + k0
                    band = ((klo <= qlo + (QB - 1) + WINDOW)
                            & (klo + (KB - 1) >= qlo - WINDOW))
                    active = band | (klo < NGLOB) | (qlo < NGLOB)

                    @pl.when(active)
                    def _():
                        row = qlo + lax.broadcasted_iota(
                            jnp.int32, (QB, KB), 0)
                        col = klo + lax.broadcasted_iota(
                            jnp.int32, (QB, KB), 1)
                        mask = ((jnp.abs(row - col) <= WINDOW)
                                | (col < NGLOB) | (row < NGLOB))
                        bias = jnp.where(mask, jnp.float32(0.0),
                                         jnp.float32(-1e9))

                        def head_step(h, _, s=s):
                            qh = qs[pl.ds(q0, QB), pl.ds(h * DH, DH)]
                            if s == 0:
                                ksrc, vsrc, roff = k_ref, v_ref, h
                            else:
                                roff = (s - 1) * HQ + h
                                ksrc, vsrc = kbuf, vbuf
                            kh = ksrc[pl.ds(roff, 1),
                                      pl.ds(k0, KB), :].reshape(KB, DH)
                            vh = vsrc[pl.ds(roff, 1),
                                      pl.ds(k0, KB), :].reshape(KB, DH)
                            sc = lax.dot_general(
                                qh, kh, (((1,), (1,)), ((), ())),
                                preferred_element_type=jnp.float32
                            ) * SCALE + bias
                            p = jnp.exp(sc)
                            l0 = l_ref[pl.ds(h, 1),
                                       pl.ds(q0, QB)].reshape(QB, 1)
                            l_ref[pl.ds(h, 1), pl.ds(q0, QB)] = (
                                l0 + jnp.sum(p, axis=1, keepdims=True)
                            ).reshape(1, QB, 1)
                            a0 = acc_ref[pl.ds(h, 1),
                                         pl.ds(q0, QB)].reshape(QB, DH)
                            acc_ref[pl.ds(h, 1), pl.ds(q0, QB)] = (
                                a0 + jnp.dot(
                                    p.astype(jnp.bfloat16), vh,
                                    preferred_element_type=jnp.float32)
                            ).reshape(1, QB, DH)
                            return 0

                        lax.fori_loop(0, HQ, head_step, 0)

                    return 0

                lax.fori_loop(0, SKV // KB, kvtile_step, 0)
                return 0

            lax.fori_loop(0, SQ // QB, qblock_step, 0)

        k1.wait_send()
        v1.wait_send()
        k3.wait_send()
        v3.wait_send()

        @pl.when(even)
        def _():
            k2s = mk(k_ref, kbuf, 2, ksend, krecv)
            v2s = mk(v_ref, vbuf, 2, vsend, vrecv)
            k2s.wait_send()
            v2s.wait_send()

        def ctx_step(h, _):
            acc = acc_ref[pl.ds(h, 1)].reshape(SQ, DH)
            l = l_ref[pl.ds(h, 1)].reshape(SQ, 1)
            qs[:, pl.ds(h * DH, DH)] = (acc / l).astype(jnp.bfloat16)
            return 0

        lax.fori_loop(0, HQ, ctx_step, 0)
        out_ref[:, :] = jnp.dot(qs[:, :], wo_ref[:, :],
                                preferred_element_type=jnp.float32)

    out2 = pl.pallas_call(
        body,
        out_shape=jax.ShapeDtypeStruct((SQ, D), jnp.float32),
        in_specs=[pl.BlockSpec(memory_space=pltpu.VMEM)] * 5,
        out_specs=pl.BlockSpec(memory_space=pltpu.VMEM),
        scratch_shapes=[
            pltpu.VMEM(((N_DEV - 1) * HQ, SKV, DH), jnp.bfloat16),
            pltpu.VMEM(((N_DEV - 1) * HQ, SKV, DH), jnp.bfloat16),
            pltpu.VMEM((SQ, D), jnp.bfloat16),
            pltpu.VMEM((HQ, SQ, 1), jnp.float32),
            pltpu.VMEM((HQ, SQ, DH), jnp.float32),
            pltpu.SemaphoreType.DMA((N_DEV - 1,)),
            pltpu.SemaphoreType.DMA((N_DEV - 1,)),
            pltpu.SemaphoreType.DMA((N_DEV - 1,)),
            pltpu.SemaphoreType.DMA((N_DEV - 1,)),
        ],
        compiler_params=pltpu.CompilerParams(
            collective_id=0, vmem_limit_bytes=44 * 1024 * 1024),
    )(x2, Wq2, Kt, Vt, Wo2)
    return out2.reshape(1, SQ, D)


# device time: 73837 ns/iter; 3.8433x vs baseline; 1.7790x over previous
import jax
import jax.numpy as jnp
from jax import lax
from jax.experimental import pallas as pl
from jax.experimental.pallas import tpu as pltpu

N_DEV = 4
SQ = 1024
SKV = 1024
HQ = 8
DH = 128
D = 1024
SCALE = 0.08838834764831843
WINDOW = 128
NGLOB = 32
HALO = 128
QB = 256
KB = 256
EXTRA_KBS = {0: (2, 3), 1: (), 2: (0,), 3: (0,)}


def kernel(x, Wq, K_ext, V_ext, Wo):
    x2 = x.reshape(SQ, D).astype(jnp.bfloat16)
    Wq2 = Wq.astype(jnp.bfloat16)
    Wo2 = Wo.astype(jnp.bfloat16)
    Kt = K_ext.reshape(SKV, HQ, DH).transpose(1, 0, 2).astype(jnp.bfloat16)
    Vt = V_ext.reshape(SKV, HQ, DH).transpose(1, 0, 2).astype(jnp.bfloat16)

    def body(x_ref, wq_ref, k_ref, v_ref, wo_ref, out_ref,
             qs, l_ref, acc_ref,
             khl, vhl, khr, vhr, kg, vg, qg, pacc, plsum, prA, prL,
             hs, hr, gsK, gsV, qgs, grK, grV, qgr,
             psA, psL, prAs, prLs):
        my = lax.axis_index("i")

        barrier = pltpu.get_barrier_semaphore()
        for d in (1, 2, 3):
            pl.semaphore_signal(barrier, inc=1,
                                device_id=(lax.rem(my + d, N_DEV),),
                                device_id_type=pl.DeviceIdType.MESH)
        pl.semaphore_wait(barrier, 3)

        l_ref[:, :, :] = jnp.zeros((HQ, SQ, 1), jnp.float32)
        acc_ref[:, :, :] = jnp.zeros((HQ, SQ, DH), jnp.float32)

        def copy(src, dst, ssem, rsem, dev):
            return pltpu.make_async_remote_copy(
                src_ref=src, dst_ref=dst, send_sem=ssem, recv_sem=rsem,
                device_id=(dev,), device_id_type=pl.DeviceIdType.MESH)

        def mk_haloR(i):
            srcs = (k_ref, v_ref)[i]
            dsts = (khl, vhl)[i]
            return copy(srcs.at[:, SKV - HALO:SKV, :], dsts.at[:, :, :],
                        hs.at[i], hr.at[i], lax.rem(my + 1, N_DEV))

        def mk_haloL(i):
            srcs = (k_ref, v_ref)[i]
            dsts = (khr, vhr)[i]
            return copy(srcs.at[:, 0:HALO, :], dsts.at[:, :, :],
                        hs.at[2 + i], hr.at[2 + i],
                        lax.rem(my + N_DEV - 1, N_DEV))

        def mk_glob(i, d):
            srcs = (k_ref, v_ref)[i]
            dsts = (kg, vg)[i]
            ss = (gsK, gsV)[i]
            rs = (grK, grV)[i]
            return copy(srcs.at[:, 0:NGLOB, :], dsts.at[:, :, :],
                        ss.at[d - 1], rs.at[0], d)

        def mk_qg(d):
            return copy(qg.at[:, :], qg.at[:, :], qgs.at[d - 1],
                        qgr.at[0], d)

        def mk_part(i):
            srcs = (pacc, plsum)[i]
            dsts = (prA, prL)[i]
            ss = (psA, psL)[i]
            rs = (prAs, prLs)[i]
            return copy(srcs.at[:, :, :],
                        dsts.at[pl.ds((my - 1) * HQ, HQ)],
                        ss.at[0], rs.at[my - 1], 0)

        @pl.when(my < N_DEV - 1)
        def _():
            for i in (0, 1):
                mk_haloR(i).start()

        @pl.when(my > 0)
        def _():
            for i in (0, 1):
                mk_haloL(i).start()

        @pl.when(my == 0)
        def _():
            qg[:, :] = jnp.dot(
                x_ref[0:NGLOB, :], wq_ref[:, :],
                preferred_element_type=jnp.float32).astype(jnp.bfloat16)
            for d in (1, 2, 3):
                mk_qg(d).start()
                for i in (0, 1):
                    mk_glob(i, d).start()

        qs[:, :] = jnp.dot(
            x_ref[:, :], wq_ref[:, :],
            preferred_element_type=jnp.float32).astype(jnp.bfloat16)

        @pl.when(my > 0)
        def _():
            mk_qg(1).wait_recv()

            def part_step(h, _):
                qh = qg[:, pl.ds(h * DH, DH)]
                kh = k_ref[pl.ds(h, 1)].reshape(SKV, DH)
                vh = v_ref[pl.ds(h, 1)].reshape(SKV, DH)
                sc = lax.dot_general(
                    qh, kh, (((1,), (1,)), ((), ())),
                    preferred_element_type=jnp.float32) * SCALE
                p = jnp.exp(sc)
                plsum[pl.ds(h, 1)] = jnp.sum(
                    p, axis=1, keepdims=True).reshape(1, NGLOB, 1)
                pacc[pl.ds(h, 1)] = jnp.dot(
                    p.astype(jnp.bfloat16), vh,
                    preferred_element_type=jnp.float32
                ).reshape(1, NGLOB, DH)
                return 0

            lax.fori_loop(0, HQ, part_step, 0)
            for i in (0, 1):
                mk_part(i).start()

        def tile(q0, kr, vr, k0, W, col0):
            row = my * SQ + q0 + lax.broadcasted_iota(jnp.int32, (QB, W), 0)
            col = col0 + lax.broadcasted_iota(jnp.int32, (QB, W), 1)
            mask = ((jnp.abs(row - col) <= WINDOW)
                    | (col < NGLOB) | (row < NGLOB))
            bias = jnp.where(mask, jnp.float32(0.0), jnp.float32(-1e9))

            def head_step(h, _):
                qh = qs[pl.ds(q0, QB), pl.ds(h * DH, DH)]
                kh = kr[pl.ds(h, 1), pl.ds(k0, W), :].reshape(W, DH)
                vh = vr[pl.ds(h, 1), pl.ds(k0, W), :].reshape(W, DH)
                sc = lax.dot_general(
                    qh, kh, (((1,), (1,)), ((), ())),
                    preferred_element_type=jnp.float32) * SCALE + bias
                p = jnp.exp(sc)
                l0 = l_ref[pl.ds(h, 1), pl.ds(q0, QB)].reshape(QB, 1)
                l_ref[pl.ds(h, 1), pl.ds(q0, QB)] = (
                    l0 + jnp.sum(p, axis=1, keepdims=True)
                ).reshape(1, QB, 1)
                a0 = acc_ref[pl.ds(h, 1), pl.ds(q0, QB)].reshape(QB, DH)
                acc_ref[pl.ds(h, 1), pl.ds(q0, QB)] = (
                    a0 + jnp.dot(p.astype(jnp.bfloat16), vh,
                                 preferred_element_type=jnp.float32)
                ).reshape(1, QB, DH)
                return 0

            lax.fori_loop(0, HQ, head_step, 0)

        for qb in range(SQ // QB):
            for kb in range(max(0, qb - 1), min(SKV // KB, qb + 2)):
                tile(qb * QB, k_ref, v_ref, kb * KB, KB,
                     my * SKV + kb * KB)
            for kb in EXTRA_KBS[qb]:
                @pl.when(my == 0)
                def _(qb=qb, kb=kb):
                    tile(qb * QB, k_ref, v_ref, kb * KB, KB,
                         my * SKV + kb * KB)

        @pl.when(my > 0)
        def _():
            for i in (0, 1):
                mk_haloR(i).wait_recv()
            tile(0, khl, vhl, 0, HALO, my * SKV - HALO)

        @pl.when(my < N_DEV - 1)
        def _():
            for i in (0, 1):
                mk_haloL(i).wait_recv()
            tile(SQ - QB, khr, vhr, 0, HALO, (my + 1) * SKV)

        @pl.when(my > 0)
        def _():
            mk_glob(0, 1).wait_recv()
            mk_glob(1, 1).wait_recv()

            def gstep(h, _):
                qh = qs[:, pl.ds(h * DH, DH)]
                kh = kg[pl.ds(h, 1)].reshape(NGLOB, DH)
                vh = vg[pl.ds(h, 1)].reshape(NGLOB, DH)
                sc = lax.dot_general(
                    qh, kh, (((1,), (1,)), ((), ())),
                    preferred_element_type=jnp.float32) * SCALE
                p = jnp.exp(sc)
                l0 = l_ref[pl.ds(h, 1)].reshape(SQ, 1)
                l_ref[pl.ds(h, 1)] = (
                    l0 + jnp.sum(p, axis=1, keepdims=True)
                ).reshape(1, SQ, 1)
                a0 = acc_ref[pl.ds(h, 1)].reshape(SQ, DH)
                acc_ref[pl.ds(h, 1)] = (
                    a0 + jnp.dot(p.astype(jnp.bfloat16), vh,
                                 preferred_element_type=jnp.float32)
                ).reshape(1, SQ, DH)
                return 0

            lax.fori_loop(0, HQ, gstep, 0)

        @pl.when(my == 0)
        def _():
            for i in (0, 1):
                for d in (1, 2, 3):
                    srcs = (pacc, plsum)[i]
                    dsts = (prA, prL)[i]
                    rs = (prAs, prLs)[i]
                    copy(srcs.at[:, :, :],
                         dsts.at[pl.ds((d - 1) * HQ, HQ)],
                         (psA, psL)[i].at[0], rs.at[d - 1], 0).wait_recv()

            def comb_step(h, _):
                a = acc_ref[pl.ds(h, 1), 0:NGLOB].reshape(NGLOB, DH)
                lsum = l_ref[pl.ds(h, 1), 0:NGLOB].reshape(NGLOB, 1)
                for d in range(3):
                    a = a + prA[pl.ds(d * HQ + h, 1)].reshape(NGLOB, DH)
                    lsum = lsum + prL[pl.ds(d * HQ + h, 1)].reshape(NGLOB, 1)
                acc_ref[pl.ds(h, 1), 0:NGLOB] = a.reshape(1, NGLOB, DH)
                l_ref[pl.ds(h, 1), 0:NGLOB] = lsum.reshape(1, NGLOB, 1)
                return 0

            lax.fori_loop(0, HQ, comb_step, 0)

        def ctx_step(h, _):
            acc = acc_ref[pl.ds(h, 1)].reshape(SQ, DH)
            l = l_ref[pl.ds(h, 1)].reshape(SQ, 1)
            qs[:, pl.ds(h * DH, DH)] = (acc / l).astype(jnp.bfloat16)
            return 0

        lax.fori_loop(0, HQ, ctx_step, 0)
        out_ref[:, :] = jnp.dot(qs[:, :], wo_ref[:, :],
                                preferred_element_type=jnp.float32)

        @pl.when(my < N_DEV - 1)
        def _():
            for i in (0, 1):
                mk_haloR(i).wait_send()

        @pl.when(my > 0)
        def _():
            for i in (0, 1):
                mk_haloL(i).wait_send()
            mk_part(0).wait_send()
            mk_part(1).wait_send()

        @pl.when(my == 0)
        def _():
            for d in (1, 2, 3):
                mk_qg(d).wait_send()
                for i in (0, 1):
                    mk_glob(i, d).wait_send()

    out2 = pl.pallas_call(
        body,
        out_shape=jax.ShapeDtypeStruct((SQ, D), jnp.float32),
        in_specs=[pl.BlockSpec(memory_space=pltpu.VMEM)] * 5,
        out_specs=pl.BlockSpec(memory_space=pltpu.VMEM),
        scratch_shapes=[
            pltpu.VMEM((SQ, D), jnp.bfloat16),
            pltpu.VMEM((HQ, SQ, 1), jnp.float32),
            pltpu.VMEM((HQ, SQ, DH), jnp.float32),
            pltpu.VMEM((HQ, HALO, DH), jnp.bfloat16),
            pltpu.VMEM((HQ, HALO, DH), jnp.bfloat16),
            pltpu.VMEM((HQ, HALO, DH), jnp.bfloat16),
            pltpu.VMEM((HQ, HALO, DH), jnp.bfloat16),
            pltpu.VMEM((HQ, NGLOB, DH), jnp.bfloat16),
            pltpu.VMEM((HQ, NGLOB, DH), jnp.bfloat16),
            pltpu.VMEM((NGLOB, D), jnp.bfloat16),
            pltpu.VMEM((HQ, NGLOB, DH), jnp.float32),
            pltpu.VMEM((HQ, NGLOB, 1), jnp.float32),
            pltpu.VMEM((3 * HQ, NGLOB, DH), jnp.float32),
            pltpu.VMEM((3 * HQ, NGLOB, 1), jnp.float32),
            pltpu.SemaphoreType.DMA((4,)),
            pltpu.SemaphoreType.DMA((4,)),
            pltpu.SemaphoreType.DMA((3,)),
            pltpu.SemaphoreType.DMA((3,)),
            pltpu.SemaphoreType.DMA((3,)),
            pltpu.SemaphoreType.DMA((1,)),
            pltpu.SemaphoreType.DMA((1,)),
            pltpu.SemaphoreType.DMA((1,)),
            pltpu.SemaphoreType.DMA((1,)),
            pltpu.SemaphoreType.DMA((1,)),
            pltpu.SemaphoreType.DMA((3,)),
            pltpu.SemaphoreType.DMA((3,)),
        ],
        compiler_params=pltpu.CompilerParams(
            collective_id=0, vmem_limit_bytes=44 * 1024 * 1024),
    )(x2, Wq2, Kt, Vt, Wo2)
    return out2.reshape(1, SQ, D)


# device time: 61035 ns/iter; 4.6495x vs baseline; 1.2097x over previous
import jax
import jax.numpy as jnp
from jax import lax
from jax.experimental import pallas as pl
from jax.experimental.pallas import tpu as pltpu

N_DEV = 4
SQ = 1024
SKV = 1024
HQ = 8
DH = 128
D = 1024
SCALE = 0.08838834764831843
WINDOW = 128
NGLOB = 32
HALO = 128
QB = 256
BW = 512


def kernel(x, Wq, K_ext, V_ext, Wo):
    x2 = x.reshape(SQ, D).astype(jnp.bfloat16)
    Wq2 = Wq.astype(jnp.bfloat16)
    Wo2 = Wo.astype(jnp.bfloat16)
    K2 = K_ext.reshape(SKV, D).astype(jnp.bfloat16)
    V2 = V_ext.reshape(SKV, D).astype(jnp.bfloat16)

    def body(x_ref, wq_ref, k_ref, v_ref, wo_ref, out_ref,
             qs, l_ref, acc_ref,
             khl, vhl, khr, vhr, kg, vg, qg, pacc, plsum, prA, prL,
             hs, hr, gsK, gsV, qgs, grK, grV, qgr,
             psA, psL, prAs, prLs):
        my = lax.axis_index("i")

        barrier = pltpu.get_barrier_semaphore()
        for d in (1, 2, 3):
            pl.semaphore_signal(barrier, inc=1,
                                device_id=(lax.rem(my + d, N_DEV),),
                                device_id_type=pl.DeviceIdType.MESH)
        pl.semaphore_wait(barrier, 3)

        def copy(src, dst, ssem, rsem, dev):
            return pltpu.make_async_remote_copy(
                src_ref=src, dst_ref=dst, send_sem=ssem, recv_sem=rsem,
                device_id=(dev,), device_id_type=pl.DeviceIdType.MESH)

        def mk_haloR(i):
            return copy((k_ref, v_ref)[i].at[pl.ds(SKV - HALO, HALO), :],
                        (khl, vhl)[i].at[:, :],
                        hs.at[i], hr.at[i], lax.rem(my + 1, N_DEV))

        def mk_haloL(i):
            return copy((k_ref, v_ref)[i].at[pl.ds(0, HALO), :],
                        (khr, vhr)[i].at[:, :],
                        hs.at[2 + i], hr.at[2 + i],
                        lax.rem(my + N_DEV - 1, N_DEV))

        def mk_glob(i, d):
            return copy((k_ref, v_ref)[i].at[pl.ds(0, NGLOB), :],
                        (kg, vg)[i].at[:, :],
                        (gsK, gsV)[i].at[d - 1], (grK, grV)[i].at[0], d)

        def mk_qg(d):
            return copy(qg.at[:, :], qg.at[:, :], qgs.at[d - 1],
                        qgr.at[0], d)

        def mk_part(i):
            dsts = (prA.at[pl.ds((my - 1) * NGLOB, NGLOB), :],
                    prL.at[pl.ds((my - 1) * HQ, HQ)])
            return copy(((pacc.at[:, :], plsum.at[:, :, :])[i]),
                        dsts[i], (psA, psL)[i].at[0],
                        (prAs, prLs)[i].at[my - 1], 0)

        @pl.when(my < N_DEV - 1)
        def _():
            for i in (0, 1):
                mk_haloR(i).start()

        @pl.when(my > 0)
        def _():
            for i in (0, 1):
                mk_haloL(i).start()

        @pl.when(my == 0)
        def _():
            qg[:, :] = jnp.dot(
                x_ref[0:NGLOB, :], wq_ref[:, :],
                preferred_element_type=jnp.float32).astype(jnp.bfloat16)
            for d in (1, 2, 3):
                mk_qg(d).start()
                for i in (0, 1):
                    mk_glob(i, d).start()
            kg[:, :] = k_ref[0:NGLOB, :]
            vg[:, :] = v_ref[0:NGLOB, :]

        qs[:, :] = jnp.dot(
            x_ref[:, :], wq_ref[:, :],
            preferred_element_type=jnp.float32).astype(jnp.bfloat16)

        @pl.when(my > 0)
        def _():
            mk_qg(1).wait_recv()

            def part_step(h, _):
                qh = qg[:, pl.ds(h * DH, DH)]
                kh = k_ref[:, pl.ds(h * DH, DH)]
                vh = v_ref[:, pl.ds(h * DH, DH)]
                sc = lax.dot_general(
                    qh, kh, (((1,), (1,)), ((), ())),
                    preferred_element_type=jnp.float32) * SCALE
                p = jnp.exp(sc)
                plsum[pl.ds(h, 1)] = jnp.sum(
                    p, axis=1, keepdims=True).reshape(1, NGLOB, 1)
                pacc[:, pl.ds(h * DH, DH)] = jnp.dot(
                    p.astype(jnp.bfloat16), vh,
                    preferred_element_type=jnp.float32)
                return 0

            lax.fori_loop(0, HQ, part_step, 0)
            for i in (0, 1):
                mk_part(i).start()
            for i in (0, 1):
                mk_glob(i, 1).wait_recv()

        def gstep(h, _):
            qh = qs[:, pl.ds(h * DH, DH)]
            kh = kg[:, pl.ds(h * DH, DH)]
            vh = vg[:, pl.ds(h * DH, DH)]
            sc = lax.dot_general(
                qh, kh, (((1,), (1,)), ((), ())),
                preferred_element_type=jnp.float32) * SCALE
            p = jnp.exp(sc)
            l_ref[pl.ds(h, 1)] = jnp.sum(
                p, axis=1, keepdims=True).reshape(1, SQ, 1)
            acc_ref[pl.ds(h, 1)] = jnp.dot(
                p.astype(jnp.bfloat16), vh,
                preferred_element_type=jnp.float32).reshape(1, SQ, DH)
            return 0

        lax.fori_loop(0, HQ, gstep, 0)

        def tile(q0, kr, vr, k0, W, col0):
            row = my * SQ + q0 + lax.broadcasted_iota(jnp.int32, (QB, W), 0)
            col = col0 + lax.broadcasted_iota(jnp.int32, (QB, W), 1)
            mask = (((jnp.abs(row - col) <= WINDOW) | (row < NGLOB))
                    & (col >= NGLOB))
            bias = jnp.where(mask, jnp.float32(0.0), jnp.float32(-1e9))

            def head_step(h, _):
                qh = qs[pl.ds(q0, QB), pl.ds(h * DH, DH)]
                kh = kr[pl.ds(k0, W), pl.ds(h * DH, DH)]
                vh = vr[pl.ds(k0, W), pl.ds(h * DH, DH)]
                sc = lax.dot_general(
                    qh, kh, (((1,), (1,)), ((), ())),
                    preferred_element_type=jnp.float32) * SCALE + bias
                p = jnp.exp(sc)
                l0 = l_ref[pl.ds(h, 1), pl.ds(q0, QB)].reshape(QB, 1)
                l_ref[pl.ds(h, 1), pl.ds(q0, QB)] = (
                    l0 + jnp.sum(p, axis=1, keepdims=True)
                ).reshape(1, QB, 1)
                a0 = acc_ref[pl.ds(h, 1), pl.ds(q0, QB)].reshape(QB, DH)
                acc_ref[pl.ds(h, 1), pl.ds(q0, QB)] = (
                    a0 + jnp.dot(p.astype(jnp.bfloat16), vh,
                                 preferred_element_type=jnp.float32)
                ).reshape(1, QB, DH)
                return 0

            lax.fori_loop(0, HQ, head_step, 0)

        for qb in range(SQ // QB):
            k0 = min(max(qb * QB - WINDOW, 0), SKV - BW)
            tile(qb * QB, k_ref, v_ref, k0, BW, my * SKV + k0)

        @pl.when(my == 0)
        def _():
            tile(0, k_ref, v_ref, BW, BW, my * SKV + BW)

        @pl.when(my > 0)
        def _():
            for i in (0, 1):
                mk_haloR(i).wait_recv()
            tile(0, khl, vhl, 0, HALO, my * SKV - HALO)

        @pl.when(my < N_DEV - 1)
        def _():
            for i in (0, 1):
                mk_haloL(i).wait_recv()
            tile(SQ - QB, khr, vhr, 0, HALO, (my + 1) * SKV)

        @pl.when(my == 0)
        def _():
            for i in (0, 1):
                for d in (1, 2, 3):
                    dsts = (prA.at[pl.ds((d - 1) * NGLOB, NGLOB), :],
                            prL.at[pl.ds((d - 1) * HQ, HQ)])
                    copy((pacc.at[:, :], plsum.at[:, :, :])[i],
                         dsts[i], (psA, psL)[i].at[0],
                         (prAs, prLs)[i].at[d - 1], 0).wait_recv()

            def comb_step(h, _):
                a = acc_ref[pl.ds(h, 1), 0:NGLOB].reshape(NGLOB, DH)
                lsum = l_ref[pl.ds(h, 1), 0:NGLOB].reshape(NGLOB, 1)
                for d in range(3):
                    a = a + prA[pl.ds(d * NGLOB, NGLOB),
                                pl.ds(h * DH, DH)]
                    lsum = lsum + prL[pl.ds(d * HQ + h, 1)].reshape(
                        NGLOB, 1)
                acc_ref[pl.ds(h, 1), 0:NGLOB] = a.reshape(1, NGLOB, DH)
                l_ref[pl.ds(h, 1), 0:NGLOB] = lsum.reshape(1, NGLOB, 1)
                return 0

            lax.fori_loop(0, HQ, comb_step, 0)

        def ctx_step(h, _):
            acc = acc_ref[pl.ds(h, 1)].reshape(SQ, DH)
            l = l_ref[pl.ds(h, 1)].reshape(SQ, 1)
            qs[:, pl.ds(h * DH, DH)] = (acc / l).astype(jnp.bfloat16)
            return 0

        lax.fori_loop(0, HQ, ctx_step, 0)
        out_ref[:, :] = jnp.dot(qs[:, :], wo_ref[:, :],
                                preferred_element_type=jnp.float32)

        @pl.when(my < N_DEV - 1)
        def _():
            for i in (0, 1):
                mk_haloR(i).wait_send()

        @pl.when(my > 0)
        def _():
            for i in (0, 1):
                mk_haloL(i).wait_send()
            mk_part(0).wait_send()
            mk_part(1).wait_send()

        @pl.when(my == 0)
        def _():
            for d in (1, 2, 3):
                mk_qg(d).wait_send()
                for i in (0, 1):
                    mk_glob(i, d).wait_send()

    out2 = pl.pallas_call(
        body,
        out_shape=jax.ShapeDtypeStruct((SQ, D), jnp.float32),
        in_specs=[pl.BlockSpec(memory_space=pltpu.VMEM)] * 5,
        out_specs=pl.BlockSpec(memory_space=pltpu.VMEM),
        scratch_shapes=[
            pltpu.VMEM((SQ, D), jnp.bfloat16),
            pltpu.VMEM((HQ, SQ, 1), jnp.float32),
            pltpu.VMEM((HQ, SQ, DH), jnp.float32),
            pltpu.VMEM((HALO, D), jnp.bfloat16),
            pltpu.VMEM((HALO, D), jnp.bfloat16),
            pltpu.VMEM((HALO, D), jnp.bfloat16),
            pltpu.VMEM((HALO, D), jnp.bfloat16),
            pltpu.VMEM((NGLOB, D), jnp.bfloat16),
            pltpu.VMEM((NGLOB, D), jnp.bfloat16),
            pltpu.VMEM((NGLOB, D), jnp.bfloat16),
            pltpu.VMEM((NGLOB, D), jnp.float32),
            pltpu.VMEM((HQ, NGLOB, 1), jnp.float32),
            pltpu.VMEM((3 * NGLOB, D), jnp.float32),
            pltpu.VMEM((3 * HQ, NGLOB, 1), jnp.float32),
            pltpu.SemaphoreType.DMA((4,)),
            pltpu.SemaphoreType.DMA((4,)),
            pltpu.SemaphoreType.DMA((3,)),
            pltpu.SemaphoreType.DMA((3,)),
            pltpu.SemaphoreType.DMA((3,)),
            pltpu.SemaphoreType.DMA((1,)),
            pltpu.SemaphoreType.DMA((1,)),
            pltpu.SemaphoreType.DMA((1,)),
            pltpu.SemaphoreType.DMA((1,)),
            pltpu.SemaphoreType.DMA((1,)),
            pltpu.SemaphoreType.DMA((3,)),
            pltpu.SemaphoreType.DMA((3,)),
        ],
        compiler_params=pltpu.CompilerParams(
            collective_id=0, vmem_limit_bytes=44 * 1024 * 1024),
    )(x2, Wq2, K2, V2, Wo2)
    return out2.reshape(1, SQ, D)


# device time: 57986 ns/iter; 4.8940x vs baseline; 1.0526x over previous
import jax
import jax.numpy as jnp
from jax import lax
from jax.experimental import pallas as pl
from jax.experimental.pallas import tpu as pltpu

N_DEV = 4
SQ = 1024
SKV = 1024
HQ = 8
DH = 128
D = 1024
SCALE = 0.08838834764831843
WINDOW = 128
NGLOB = 32
HALO = 128
QB = 256
BW = 512


def kernel(x, Wq, K_ext, V_ext, Wo):
    x2 = x.reshape(SQ, D)
    K2 = K_ext.reshape(SKV, D)
    V2 = V_ext.reshape(SKV, D)

    def body(x_ref, wq_ref, k_ref, v_ref, wo_ref, out_ref,
             qs, l_ref, acc_ref, kb, vb,
             khl, vhl, khr, vhr, kg, vg, qg, pacc, plsum, prA, prL,
             hs, hr, gsK, gsV, qgs, grK, grV, qgr,
             psA, psL, prAs, prLs):
        my = lax.axis_index("i")

        barrier = pltpu.get_barrier_semaphore()
        for d in (1, 2, 3):
            pl.semaphore_signal(barrier, inc=1,
                                device_id=(lax.rem(my + d, N_DEV),),
                                device_id_type=pl.DeviceIdType.MESH)
        pl.semaphore_wait(barrier, 3)

        kb[:, :] = k_ref[:, :].astype(jnp.bfloat16)
        vb[:, :] = v_ref[:, :].astype(jnp.bfloat16)

        def copy(src, dst, ssem, rsem, dev):
            return pltpu.make_async_remote_copy(
                src_ref=src, dst_ref=dst, send_sem=ssem, recv_sem=rsem,
                device_id=(dev,), device_id_type=pl.DeviceIdType.MESH)

        def mk_haloR(i):
            return copy((kb, vb)[i].at[pl.ds(SKV - HALO, HALO), :],
                        (khl, vhl)[i].at[:, :],
                        hs.at[i], hr.at[i], lax.rem(my + 1, N_DEV))

        def mk_haloL(i):
            return copy((kb, vb)[i].at[pl.ds(0, HALO), :],
                        (khr, vhr)[i].at[:, :],
                        hs.at[2 + i], hr.at[2 + i],
                        lax.rem(my + N_DEV - 1, N_DEV))

        def mk_glob(i, d):
            return copy((kb, vb)[i].at[pl.ds(0, NGLOB), :],
                        (kg, vg)[i].at[:, :],
                        (gsK, gsV)[i].at[d - 1], (grK, grV)[i].at[0], d)

        def mk_qg(d):
            return copy(qg.at[:, :], qg.at[:, :], qgs.at[d - 1],
                        qgr.at[0], d)

        def mk_part(i):
            dsts = (prA.at[pl.ds((my - 1) * NGLOB, NGLOB), :],
                    prL.at[pl.ds((my - 1) * HQ, HQ)])
            return copy(((pacc.at[:, :], plsum.at[:, :, :])[i]),
                        dsts[i], (psA, psL)[i].at[0],
                        (prAs, prLs)[i].at[my - 1], 0)

        @pl.when(my < N_DEV - 1)
        def _():
            for i in (0, 1):
                mk_haloR(i).start()

        @pl.when(my > 0)
        def _():
            for i in (0, 1):
                mk_haloL(i).start()

        @pl.when(my == 0)
        def _():
            for d in (1, 2, 3):
                for i in (0, 1):
                    mk_glob(i, d).start()
            kg[:, :] = kb[0:NGLOB, :]
            vg[:, :] = vb[0:NGLOB, :]

        qs[:, :] = jnp.dot(
            x_ref[:, :].astype(jnp.bfloat16),
            wq_ref[:, :].astype(jnp.bfloat16),
            preferred_element_type=jnp.float32).astype(jnp.bfloat16)

        @pl.when(my == 0)
        def _():
            qg[:, :] = qs[0:NGLOB, :]
            for d in (1, 2, 3):
                mk_qg(d).start()

        @pl.when(my > 0)
        def _():
            mk_qg(1).wait_recv()

            def part_step(h, _):
                qh = qg[:, pl.ds(h * DH, DH)]
                kh = kb[:, pl.ds(h * DH, DH)]
                vh = vb[:, pl.ds(h * DH, DH)]
                sc = lax.dot_general(
                    qh, kh, (((1,), (1,)), ((), ())),
                    preferred_element_type=jnp.float32) * SCALE
                p = jnp.exp(sc)
                plsum[pl.ds(h, 1)] = jnp.sum(
                    p, axis=1, keepdims=True).reshape(1, NGLOB, 1)
                pacc[:, pl.ds(h * DH, DH)] = jnp.dot(
                    p.astype(jnp.bfloat16), vh,
                    preferred_element_type=jnp.float32)
                return 0

            lax.fori_loop(0, HQ, part_step, 0)
            for i in (0, 1):
                mk_part(i).start()
            for i in (0, 1):
                mk_glob(i, 1).wait_recv()

        def gstep(h, _):
            qh = qs[:, pl.ds(h * DH, DH)]
            kh = kg[:, pl.ds(h * DH, DH)]
            vh = vg[:, pl.ds(h * DH, DH)]
            sc = lax.dot_general(
                qh, kh, (((1,), (1,)), ((), ())),
                preferred_element_type=jnp.float32) * SCALE
            p = jnp.exp(sc)
            l_ref[pl.ds(h, 1)] = jnp.sum(
                p, axis=1, keepdims=True).reshape(1, SQ, 1)
            acc_ref[pl.ds(h, 1)] = jnp.dot(
                p.astype(jnp.bfloat16), vh,
                preferred_element_type=jnp.float32).reshape(1, SQ, DH)
            return 0

        lax.fori_loop(0, HQ, gstep, 0)

        def tile(q0, kr, vr, k0, W, col0):
            row = my * SQ + q0 + lax.broadcasted_iota(jnp.int32, (QB, W), 0)
            col = col0 + lax.broadcasted_iota(jnp.int32, (QB, W), 1)
            mask = (((jnp.abs(row - col) <= WINDOW) | (row < NGLOB))
                    & (col >= NGLOB))
            bias = jnp.where(mask, jnp.float32(0.0), jnp.float32(-1e9))

            def head_step(h, _):
                qh = qs[pl.ds(q0, QB), pl.ds(h * DH, DH)]
                kh = kr[pl.ds(k0, W), pl.ds(h * DH, DH)]
                vh = vr[pl.ds(k0, W), pl.ds(h * DH, DH)]
                sc = lax.dot_general(
                    qh, kh, (((1,), (1,)), ((), ())),
                    preferred_element_type=jnp.float32) * SCALE + bias
                p = jnp.exp(sc)
                l0 = l_ref[pl.ds(h, 1), pl.ds(q0, QB)].reshape(QB, 1)
                l_ref[pl.ds(h, 1), pl.ds(q0, QB)] = (
                    l0 + jnp.sum(p, axis=1, keepdims=True)
                ).reshape(1, QB, 1)
                a0 = acc_ref[pl.ds(h, 1), pl.ds(q0, QB)].reshape(QB, DH)
                acc_ref[pl.ds(h, 1), pl.ds(q0, QB)] = (
                    a0 + jnp.dot(p.astype(jnp.bfloat16), vh,
                                 preferred_element_type=jnp.float32)
                ).reshape(1, QB, DH)
                return 0

            lax.fori_loop(0, HQ, head_step, 0)

        for qb in range(SQ // QB):
            k0 = min(max(qb * QB - WINDOW, 0), SKV - BW)
            tile(qb * QB, kb, vb, k0, BW, my * SKV + k0)

        @pl.when(my == 0)
        def _():
            tile(0, kb, vb, BW, BW, my * SKV + BW)

        @pl.when(my > 0)
        def _():
            for i in (0, 1):
                mk_haloR(i).wait_recv()
            tile(0, khl, vhl, 0, HALO, my * SKV - HALO)

        @pl.when(my < N_DEV - 1)
        def _():
            for i in (0, 1):
                mk_haloL(i).wait_recv()
            tile(SQ - QB, khr, vhr, 0, HALO, (my + 1) * SKV)

        @pl.when(my == 0)
        def _():
            for i in (0, 1):
                for d in (1, 2, 3):
                    dsts = (prA.at[pl.ds((d - 1) * NGLOB, NGLOB), :],
                            prL.at[pl.ds((d - 1) * HQ, HQ)])
                    copy((pacc.at[:, :], plsum.at[:, :, :])[i],
                         dsts[i], (psA, psL)[i].at[0],
                         (prAs, prLs)[i].at[d - 1], 0).wait_recv()

            def comb_step(h, _):
                a = acc_ref[pl.ds(h, 1), 0:NGLOB].reshape(NGLOB, DH)
                lsum = l_ref[pl.ds(h, 1), 0:NGLOB].reshape(NGLOB, 1)
                for d in range(3):
                    a = a + prA[pl.ds(d * NGLOB, NGLOB),
                                pl.ds(h * DH, DH)]
                    lsum = lsum + prL[pl.ds(d * HQ + h, 1)].reshape(
                        NGLOB, 1)
                acc_ref[pl.ds(h, 1), 0:NGLOB] = a.reshape(1, NGLOB, DH)
                l_ref[pl.ds(h, 1), 0:NGLOB] = lsum.reshape(1, NGLOB, 1)
                return 0

            lax.fori_loop(0, HQ, comb_step, 0)

        def ctx_step(h, _):
            acc = acc_ref[pl.ds(h, 1)].reshape(SQ, DH)
            l = l_ref[pl.ds(h, 1)].reshape(SQ, 1)
            qs[:, pl.ds(h * DH, DH)] = (acc / l).astype(jnp.bfloat16)
            return 0

        lax.fori_loop(0, HQ, ctx_step, 0)
        out_ref[:, :] = jnp.dot(qs[:, :],
                                wo_ref[:, :].astype(jnp.bfloat16),
                                preferred_element_type=jnp.float32)

        @pl.when(my < N_DEV - 1)
        def _():
            for i in (0, 1):
                mk_haloR(i).wait_send()

        @pl.when(my > 0)
        def _():
            for i in (0, 1):
                mk_haloL(i).wait_send()
            mk_part(0).wait_send()
            mk_part(1).wait_send()

        @pl.when(my == 0)
        def _():
            for d in (1, 2, 3):
                mk_qg(d).wait_send()
                for i in (0, 1):
                    mk_glob(i, d).wait_send()

    out2 = pl.pallas_call(
        body,
        out_shape=jax.ShapeDtypeStruct((SQ, D), jnp.float32),
        in_specs=[pl.BlockSpec(memory_space=pltpu.VMEM)] * 5,
        out_specs=pl.BlockSpec(memory_space=pltpu.VMEM),
        scratch_shapes=[
            pltpu.VMEM((SQ, D), jnp.bfloat16),
            pltpu.VMEM((HQ, SQ, 1), jnp.float32),
            pltpu.VMEM((HQ, SQ, DH), jnp.float32),
            pltpu.VMEM((SKV, D), jnp.bfloat16),
            pltpu.VMEM((SKV, D), jnp.bfloat16),
            pltpu.VMEM((HALO, D), jnp.bfloat16),
            pltpu.VMEM((HALO, D), jnp.bfloat16),
            pltpu.VMEM((HALO, D), jnp.bfloat16),
            pltpu.VMEM((HALO, D), jnp.bfloat16),
            pltpu.VMEM((NGLOB, D), jnp.bfloat16),
            pltpu.VMEM((NGLOB, D), jnp.bfloat16),
            pltpu.VMEM((NGLOB, D), jnp.bfloat16),
            pltpu.VMEM((NGLOB, D), jnp.float32),
            pltpu.VMEM((HQ, NGLOB, 1), jnp.float32),
            pltpu.VMEM((3 * NGLOB, D), jnp.float32),
            pltpu.VMEM((3 * HQ, NGLOB, 1), jnp.float32),
            pltpu.SemaphoreType.DMA((4,)),
            pltpu.SemaphoreType.DMA((4,)),
            pltpu.SemaphoreType.DMA((3,)),
            pltpu.SemaphoreType.DMA((3,)),
            pltpu.SemaphoreType.DMA((3,)),
            pltpu.SemaphoreType.DMA((1,)),
            pltpu.SemaphoreType.DMA((1,)),
            pltpu.SemaphoreType.DMA((1,)),
            pltpu.SemaphoreType.DMA((1,)),
            pltpu.SemaphoreType.DMA((1,)),
            pltpu.SemaphoreType.DMA((3,)),
            pltpu.SemaphoreType.DMA((3,)),
        ],
        compiler_params=pltpu.CompilerParams(
            collective_id=0, vmem_limit_bytes=44 * 1024 * 1024),
    )(x2, Wq, K2, V2, Wo)
    return out2.reshape(1, SQ, D)


# device time: 57678 ns/iter; 4.9201x vs baseline; 1.0053x over previous
import jax
import jax.numpy as jnp
from jax import lax
from jax.experimental import pallas as pl
from jax.experimental.pallas import tpu as pltpu

N_DEV = 4
SQ = 1024
SKV = 1024
HQ = 8
DH = 128
D = 1024
SCALE = 0.08838834764831843
WINDOW = 128
NGLOB = 32
HALO = 128
QB = 256
BW = 512


def kernel(x, Wq, K_ext, V_ext, Wo):
    x2 = x.reshape(SQ, D)
    Kt = K_ext.reshape(SKV, HQ, DH).transpose(1, 0, 2)
    Vt = V_ext.reshape(SKV, HQ, DH).transpose(1, 0, 2)

    def body(x_ref, wq_ref, k_ref, v_ref, wo_ref, out_ref,
             qs, l_ref, acc_ref, kb, vb,
             khl, vhl, khr, vhr, kg, vg, qg, pacc, plsum, prA, prL,
             hs, hr, gsK, gsV, qgs, grK, grV, qgr,
             psA, psL, prAs, prLs):
        my = lax.axis_index("i")

        barrier = pltpu.get_barrier_semaphore()
        for d in (1, 2, 3):
            pl.semaphore_signal(barrier, inc=1,
                                device_id=(lax.rem(my + d, N_DEV),),
                                device_id_type=pl.DeviceIdType.MESH)
        pl.semaphore_wait(barrier, 3)

        kb[:, :, :] = k_ref[:, :, :].astype(jnp.bfloat16)
        vb[:, :, :] = v_ref[:, :, :].astype(jnp.bfloat16)

        def copy(src, dst, ssem, rsem, dev):
            return pltpu.make_async_remote_copy(
                src_ref=src, dst_ref=dst, send_sem=ssem, recv_sem=rsem,
                device_id=(dev,), device_id_type=pl.DeviceIdType.MESH)

        def mk_haloR(i):
            return copy((kb, vb)[i].at[:, pl.ds(SKV - HALO, HALO), :],
                        (khl, vhl)[i].at[:, :, :],
                        hs.at[i], hr.at[i], lax.rem(my + 1, N_DEV))

        def mk_haloL(i):
            return copy((kb, vb)[i].at[:, pl.ds(0, HALO), :],
                        (khr, vhr)[i].at[:, :, :],
                        hs.at[2 + i], hr.at[2 + i],
                        lax.rem(my + N_DEV - 1, N_DEV))

        def mk_glob(i, d):
            return copy((kb, vb)[i].at[:, pl.ds(0, NGLOB), :],
                        (kg, vg)[i].at[:, :, :],
                        (gsK, gsV)[i].at[d - 1], (grK, grV)[i].at[0], d)

        def mk_qg(d):
            return copy(qg.at[:, :], qg.at[:, :], qgs.at[d - 1],
                        qgr.at[0], d)

        def mk_part(i):
            dsts = (prA.at[pl.ds((my - 1) * NGLOB, NGLOB), :],
                    prL.at[pl.ds((my - 1) * HQ, HQ)])
            return copy(((pacc.at[:, :], plsum.at[:, :, :])[i]),
                        dsts[i], (psA, psL)[i].at[0],
                        (prAs, prLs)[i].at[my - 1], 0)

        @pl.when(my < N_DEV - 1)
        def _():
            for i in (0, 1):
                mk_haloR(i).start()

        @pl.when(my > 0)
        def _():
            for i in (0, 1):
                mk_haloL(i).start()

        @pl.when(my == 0)
        def _():
            for d in (1, 2, 3):
                for i in (0, 1):
                    mk_glob(i, d).start()
            kg[:, :, :] = kb[:, 0:NGLOB, :]
            vg[:, :, :] = vb[:, 0:NGLOB, :]

        qs[:, :] = jnp.dot(
            x_ref[:, :].astype(jnp.bfloat16),
            wq_ref[:, :].astype(jnp.bfloat16),
            preferred_element_type=jnp.float32).astype(jnp.bfloat16)

        @pl.when(my == 0)
        def _():
            qg[:, :] = qs[0:NGLOB, :]
            for d in (1, 2, 3):
                mk_qg(d).start()

        @pl.when(my > 0)
        def _():
            mk_qg(1).wait_recv()

            def part_step(h, _):
                qh = qg[:, pl.ds(h * DH, DH)]
                kh = kb[pl.ds(h, 1)].reshape(SKV, DH)
                vh = vb[pl.ds(h, 1)].reshape(SKV, DH)
                sc = lax.dot_general(
                    qh, kh, (((1,), (1,)), ((), ())),
                    preferred_element_type=jnp.float32) * SCALE
                p = jnp.exp(sc)
                plsum[pl.ds(h, 1)] = jnp.sum(
                    p, axis=1, keepdims=True).reshape(1, NGLOB, 1)
                pacc[:, pl.ds(h * DH, DH)] = jnp.dot(
                    p.astype(jnp.bfloat16), vh,
                    preferred_element_type=jnp.float32)
                return 0

            lax.fori_loop(0, HQ, part_step, 0)
            for i in (0, 1):
                mk_part(i).start()
            for i in (0, 1):
                mk_glob(i, 1).wait_recv()

        def gstep(h, _):
            qh = qs[:, pl.ds(h * DH, DH)]
            kh = kg[pl.ds(h, 1)].reshape(NGLOB, DH)
            vh = vg[pl.ds(h, 1)].reshape(NGLOB, DH)
            sc = lax.dot_general(
                qh, kh, (((1,), (1,)), ((), ())),
                preferred_element_type=jnp.float32) * SCALE
            p = jnp.exp(sc)
            l_ref[pl.ds(h, 1)] = jnp.sum(
                p, axis=1, keepdims=True).reshape(1, SQ, 1)
            acc_ref[pl.ds(h, 1)] = jnp.dot(
                p.astype(jnp.bfloat16), vh,
                preferred_element_type=jnp.float32).reshape(1, SQ, DH)
            return 0

        lax.fori_loop(0, HQ, gstep, 0)

        def tile(q0, kr, vr, k0, W, col0):
            row = my * SQ + q0 + lax.broadcasted_iota(jnp.int32, (QB, W), 0)
            col = col0 + lax.broadcasted_iota(jnp.int32, (QB, W), 1)
            mask = (((jnp.abs(row - col) <= WINDOW) | (row < NGLOB))
                    & (col >= NGLOB))
            bias = jnp.where(mask, jnp.float32(0.0), jnp.float32(-1e9))

            def head_step(h, _):
                qh = qs[pl.ds(q0, QB), pl.ds(h * DH, DH)]
                kh = kr[pl.ds(h, 1), pl.ds(k0, W), :].reshape(W, DH)
                vh = vr[pl.ds(h, 1), pl.ds(k0, W), :].reshape(W, DH)
                sc = lax.dot_general(
                    qh, kh, (((1,), (1,)), ((), ())),
                    preferred_element_type=jnp.float32) * SCALE + bias
                p = jnp.exp(sc)
                l0 = l_ref[pl.ds(h, 1), pl.ds(q0, QB)].reshape(QB, 1)
                l_ref[pl.ds(h, 1), pl.ds(q0, QB)] = (
                    l0 + jnp.sum(p, axis=1, keepdims=True)
                ).reshape(1, QB, 1)
                a0 = acc_ref[pl.ds(h, 1), pl.ds(q0, QB)].reshape(QB, DH)
                acc_ref[pl.ds(h, 1), pl.ds(q0, QB)] = (
                    a0 + jnp.dot(p.astype(jnp.bfloat16), vh,
                                 preferred_element_type=jnp.float32)
                ).reshape(1, QB, DH)
                return 0

            lax.fori_loop(0, HQ, head_step, 0)

        for qb in range(SQ // QB):
            k0 = min(max(qb * QB - WINDOW, 0), SKV - BW)
            tile(qb * QB, kb, vb, k0, BW, my * SKV + k0)

        @pl.when(my == 0)
        def _():
            tile(0, kb, vb, BW, BW, my * SKV + BW)

        @pl.when(my > 0)
        def _():
            for i in (0, 1):
                mk_haloR(i).wait_recv()
            tile(0, khl, vhl, 0, HALO, my * SKV - HALO)

        @pl.when(my < N_DEV - 1)
        def _():
            for i in (0, 1):
                mk_haloL(i).wait_recv()
            tile(SQ - QB, khr, vhr, 0, HALO, (my + 1) * SKV)

        @pl.when(my == 0)
        def _():
            for i in (0, 1):
                for d in (1, 2, 3):
                    dsts = (prA.at[pl.ds((d - 1) * NGLOB, NGLOB), :],
                            prL.at[pl.ds((d - 1) * HQ, HQ)])
                    copy((pacc.at[:, :], plsum.at[:, :, :])[i],
                         dsts[i], (psA, psL)[i].at[0],
                         (prAs, prLs)[i].at[d - 1], 0).wait_recv()

            def comb_step(h, _):
                a = acc_ref[pl.ds(h, 1), 0:NGLOB].reshape(NGLOB, DH)
                lsum = l_ref[pl.ds(h, 1), 0:NGLOB].reshape(NGLOB, 1)
                for d in range(3):
                    a = a + prA[pl.ds(d * NGLOB, NGLOB),
                                pl.ds(h * DH, DH)]
                    lsum = lsum + prL[pl.ds(d * HQ + h, 1)].reshape(
                        NGLOB, 1)
                acc_ref[pl.ds(h, 1), 0:NGLOB] = a.reshape(1, NGLOB, DH)
                l_ref[pl.ds(h, 1), 0:NGLOB] = lsum.reshape(1, NGLOB, 1)
                return 0

            lax.fori_loop(0, HQ, comb_step, 0)

        def ctx_step(h, _):
            acc = acc_ref[pl.ds(h, 1)].reshape(SQ, DH)
            l = l_ref[pl.ds(h, 1)].reshape(SQ, 1)
            qs[:, pl.ds(h * DH, DH)] = (acc / l).astype(jnp.bfloat16)
            return 0

        lax.fori_loop(0, HQ, ctx_step, 0)
        out_ref[:, :] = jnp.dot(qs[:, :],
                                wo_ref[:, :].astype(jnp.bfloat16),
                                preferred_element_type=jnp.float32)

        @pl.when(my < N_DEV - 1)
        def _():
            for i in (0, 1):
                mk_haloR(i).wait_send()

        @pl.when(my > 0)
        def _():
            for i in (0, 1):
                mk_haloL(i).wait_send()
            mk_part(0).wait_send()
            mk_part(1).wait_send()

        @pl.when(my == 0)
        def _():
            for d in (1, 2, 3):
                mk_qg(d).wait_send()
                for i in (0, 1):
                    mk_glob(i, d).wait_send()

    out2 = pl.pallas_call(
        body,
        out_shape=jax.ShapeDtypeStruct((SQ, D), jnp.float32),
        in_specs=[pl.BlockSpec(memory_space=pltpu.VMEM)] * 5,
        out_specs=pl.BlockSpec(memory_space=pltpu.VMEM),
        scratch_shapes=[
            pltpu.VMEM((SQ, D), jnp.bfloat16),
            pltpu.VMEM((HQ, SQ, 1), jnp.float32),
            pltpu.VMEM((HQ, SQ, DH), jnp.float32),
            pltpu.VMEM((HQ, SKV, DH), jnp.bfloat16),
            pltpu.VMEM((HQ, SKV, DH), jnp.bfloat16),
            pltpu.VMEM((HQ, HALO, DH), jnp.bfloat16),
            pltpu.VMEM((HQ, HALO, DH), jnp.bfloat16),
            pltpu.VMEM((HQ, HALO, DH), jnp.bfloat16),
            pltpu.VMEM((HQ, HALO, DH), jnp.bfloat16),
            pltpu.VMEM((HQ, NGLOB, DH), jnp.bfloat16),
            pltpu.VMEM((HQ, NGLOB, DH), jnp.bfloat16),
            pltpu.VMEM((NGLOB, D), jnp.bfloat16),
            pltpu.VMEM((NGLOB, D), jnp.float32),
            pltpu.VMEM((HQ, NGLOB, 1), jnp.float32),
            pltpu.VMEM((3 * NGLOB, D), jnp.float32),
            pltpu.VMEM((3 * HQ, NGLOB, 1), jnp.float32),
            pltpu.SemaphoreType.DMA((4,)),
            pltpu.SemaphoreType.DMA((4,)),
            pltpu.SemaphoreType.DMA((3,)),
            pltpu.SemaphoreType.DMA((3,)),
            pltpu.SemaphoreType.DMA((3,)),
            pltpu.SemaphoreType.DMA((1,)),
            pltpu.SemaphoreType.DMA((1,)),
            pltpu.SemaphoreType.DMA((1,)),
            pltpu.SemaphoreType.DMA((1,)),
            pltpu.SemaphoreType.DMA((1,)),
            pltpu.SemaphoreType.DMA((3,)),
            pltpu.SemaphoreType.DMA((3,)),
        ],
        compiler_params=pltpu.CompilerParams(
            collective_id=0, vmem_limit_bytes=44 * 1024 * 1024),
    )(x2, Wq, Kt, Vt, Wo)
    return out2.reshape(1, SQ, D)


# device time: 53515 ns/iter; 5.3028x vs baseline; 1.0778x over previous
import jax
import jax.numpy as jnp
from jax import lax
from jax.experimental import pallas as pl
from jax.experimental.pallas import tpu as pltpu

N_DEV = 4
SQ = 1024
SKV = 1024
HQ = 8
DH = 128
D = 1024
SCALE = 0.08838834764831843
WINDOW = 128
NGLOB = 32
HALO = 128
QB = 256
BW = 512


def kernel(x, Wq, K_ext, V_ext, Wo):
    x2 = x.reshape(SQ, D)
    Kt = K_ext.reshape(SKV, HQ, DH).transpose(1, 0, 2).astype(jnp.bfloat16)
    Vt = V_ext.reshape(SKV, HQ, DH).transpose(1, 0, 2).astype(jnp.bfloat16)

    def body(x_ref, wq_ref, k_ref, v_ref, wo_ref, out_ref,
             qs, l_ref, acc_ref,
             khl, vhl, khr, vhr, kg, vg, qg, pacc, plsum, prA, prL,
             hs, hr, gsK, gsV, qgs, grK, grV, qgr,
             psA, psL, prAs, prLs):
        my = lax.axis_index("i")

        barrier = pltpu.get_barrier_semaphore()
        for d in (1, 2, 3):
            pl.semaphore_signal(barrier, inc=1,
                                device_id=(lax.rem(my + d, N_DEV),),
                                device_id_type=pl.DeviceIdType.MESH)
        pl.semaphore_wait(barrier, 3)

        kb, vb = k_ref, v_ref

        def copy(src, dst, ssem, rsem, dev):
            return pltpu.make_async_remote_copy(
                src_ref=src, dst_ref=dst, send_sem=ssem, recv_sem=rsem,
                device_id=(dev,), device_id_type=pl.DeviceIdType.MESH)

        def mk_haloR(i):
            return copy((kb, vb)[i].at[:, pl.ds(SKV - HALO, HALO), :],
                        (khl, vhl)[i].at[:, :, :],
                        hs.at[i], hr.at[i], lax.rem(my + 1, N_DEV))

        def mk_haloL(i):
            return copy((kb, vb)[i].at[:, pl.ds(0, HALO), :],
                        (khr, vhr)[i].at[:, :, :],
                        hs.at[2 + i], hr.at[2 + i],
                        lax.rem(my + N_DEV - 1, N_DEV))

        def mk_glob(i, d):
            return copy((kb, vb)[i].at[:, pl.ds(0, NGLOB), :],
                        (kg, vg)[i].at[:, :, :],
                        (gsK, gsV)[i].at[d - 1], (grK, grV)[i].at[0], d)

        def mk_qg(d):
            return copy(qg.at[:, :], qg.at[:, :], qgs.at[d - 1],
                        qgr.at[0], d)

        def mk_part(i):
            dsts = (prA.at[pl.ds((my - 1) * NGLOB, NGLOB), :],
                    prL.at[pl.ds((my - 1) * HQ, HQ)])
            return copy(((pacc.at[:, :], plsum.at[:, :, :])[i]),
                        dsts[i], (psA, psL)[i].at[0],
                        (prAs, prLs)[i].at[my - 1], 0)

        @pl.when(my < N_DEV - 1)
        def _():
            for i in (0, 1):
                mk_haloR(i).start()

        @pl.when(my > 0)
        def _():
            for i in (0, 1):
                mk_haloL(i).start()

        @pl.when(my == 0)
        def _():
            for d in (1, 2, 3):
                for i in (0, 1):
                    mk_glob(i, d).start()
            kg[:, :, :] = kb[:, 0:NGLOB, :]
            vg[:, :, :] = vb[:, 0:NGLOB, :]

        qs[:, :] = jnp.dot(
            x_ref[:, :].astype(jnp.bfloat16),
            wq_ref[:, :].astype(jnp.bfloat16),
            preferred_element_type=jnp.float32).astype(jnp.bfloat16)

        @pl.when(my == 0)
        def _():
            qg[:, :] = qs[0:NGLOB, :]
            for d in (1, 2, 3):
                mk_qg(d).start()

        @pl.when(my > 0)
        def _():
            mk_qg(1).wait_recv()

            def part_step(h, _):
                qh = qg[:, pl.ds(h * DH, DH)]
                kh = kb[pl.ds(h, 1)].reshape(SKV, DH)
                vh = vb[pl.ds(h, 1)].reshape(SKV, DH)
                sc = lax.dot_general(
                    qh, kh, (((1,), (1,)), ((), ())),
                    preferred_element_type=jnp.float32) * SCALE
                p = jnp.exp(sc)
                plsum[pl.ds(h, 1)] = jnp.sum(
                    p, axis=1, keepdims=True).reshape(1, NGLOB, 1)
                pacc[:, pl.ds(h * DH, DH)] = jnp.dot(
                    p.astype(jnp.bfloat16), vh,
                    preferred_element_type=jnp.float32)
                return 0

            lax.fori_loop(0, HQ, part_step, 0)
            for i in (0, 1):
                mk_part(i).start()
            for i in (0, 1):
                mk_glob(i, 1).wait_recv()

        def gstep(h, _):
            qh = qs[:, pl.ds(h * DH, DH)]
            kh = kg[pl.ds(h, 1)].reshape(NGLOB, DH)
            vh = vg[pl.ds(h, 1)].reshape(NGLOB, DH)
            sc = lax.dot_general(
                qh, kh, (((1,), (1,)), ((), ())),
                preferred_element_type=jnp.float32) * SCALE
            p = jnp.exp(sc)
            l_ref[pl.ds(h, 1)] = jnp.sum(
                p, axis=1, keepdims=True).reshape(1, SQ, 1)
            acc_ref[pl.ds(h, 1)] = jnp.dot(
                p.astype(jnp.bfloat16), vh,
                preferred_element_type=jnp.float32).reshape(1, SQ, DH)
            return 0

        lax.fori_loop(0, HQ, gstep, 0)

        def tile(q0, kr, vr, k0, W, col0):
            row = my * SQ + q0 + lax.broadcasted_iota(jnp.int32, (QB, W), 0)
            col = col0 + lax.broadcasted_iota(jnp.int32, (QB, W), 1)
            mask = (((jnp.abs(row - col) <= WINDOW) | (row < NGLOB))
                    & (col >= NGLOB))
            bias = jnp.where(mask, jnp.float32(0.0), jnp.float32(-1e9))

            def head_step(h, _):
                qh = qs[pl.ds(q0, QB), pl.ds(h * DH, DH)]
                kh = kr[pl.ds(h, 1), pl.ds(k0, W), :].reshape(W, DH)
                vh = vr[pl.ds(h, 1), pl.ds(k0, W), :].reshape(W, DH)
                sc = lax.dot_general(
                    qh, kh, (((1,), (1,)), ((), ())),
                    preferred_element_type=jnp.float32) * SCALE + bias
                p = jnp.exp(sc)
                l0 = l_ref[pl.ds(h, 1), pl.ds(q0, QB)].reshape(QB, 1)
                l_ref[pl.ds(h, 1), pl.ds(q0, QB)] = (
                    l0 + jnp.sum(p, axis=1, keepdims=True)
                ).reshape(1, QB, 1)
                a0 = acc_ref[pl.ds(h, 1), pl.ds(q0, QB)].reshape(QB, DH)
                acc_ref[pl.ds(h, 1), pl.ds(q0, QB)] = (
                    a0 + jnp.dot(p.astype(jnp.bfloat16), vh,
                                 preferred_element_type=jnp.float32)
                ).reshape(1, QB, DH)
                return 0

            lax.fori_loop(0, HQ, head_step, 0)

        for qb in range(SQ // QB):
            k0 = min(max(qb * QB - WINDOW, 0), SKV - BW)
            tile(qb * QB, kb, vb, k0, BW, my * SKV + k0)

        @pl.when(my == 0)
        def _():
            tile(0, kb, vb, BW, BW, my * SKV + BW)

        @pl.when(my > 0)
        def _():
            for i in (0, 1):
                mk_haloR(i).wait_recv()
            tile(0, khl, vhl, 0, HALO, my * SKV - HALO)

        @pl.when(my < N_DEV - 1)
        def _():
            for i in (0, 1):
                mk_haloL(i).wait_recv()
            tile(SQ - QB, khr, vhr, 0, HALO, (my + 1) * SKV)

        @pl.when(my == 0)
        def _():
            for i in (0, 1):
                for d in (1, 2, 3):
                    dsts = (prA.at[pl.ds((d - 1) * NGLOB, NGLOB), :],
                            prL.at[pl.ds((d - 1) * HQ, HQ)])
                    copy((pacc.at[:, :], plsum.at[:, :, :])[i],
                         dsts[i], (psA, psL)[i].at[0],
                         (prAs, prLs)[i].at[d - 1], 0).wait_recv()

            def comb_step(h, _):
                a = acc_ref[pl.ds(h, 1), 0:NGLOB].reshape(NGLOB, DH)
                lsum = l_ref[pl.ds(h, 1), 0:NGLOB].reshape(NGLOB, 1)
                for d in range(3):
                    a = a + prA[pl.ds(d * NGLOB, NGLOB),
                                pl.ds(h * DH, DH)]
                    lsum = lsum + prL[pl.ds(d * HQ + h, 1)].reshape(
                        NGLOB, 1)
                acc_ref[pl.ds(h, 1), 0:NGLOB] = a.reshape(1, NGLOB, DH)
                l_ref[pl.ds(h, 1), 0:NGLOB] = lsum.reshape(1, NGLOB, 1)
                return 0

            lax.fori_loop(0, HQ, comb_step, 0)

        def ctx_step(h, _):
            acc = acc_ref[pl.ds(h, 1)].reshape(SQ, DH)
            l = l_ref[pl.ds(h, 1)].reshape(SQ, 1)
            qs[:, pl.ds(h * DH, DH)] = (acc / l).astype(jnp.bfloat16)
            return 0

        lax.fori_loop(0, HQ, ctx_step, 0)
        out_ref[:, :] = jnp.dot(qs[:, :],
                                wo_ref[:, :].astype(jnp.bfloat16),
                                preferred_element_type=jnp.float32)

        @pl.when(my < N_DEV - 1)
        def _():
            for i in (0, 1):
                mk_haloR(i).wait_send()

        @pl.when(my > 0)
        def _():
            for i in (0, 1):
                mk_haloL(i).wait_send()
            mk_part(0).wait_send()
            mk_part(1).wait_send()

        @pl.when(my == 0)
        def _():
            for d in (1, 2, 3):
                mk_qg(d).wait_send()
                for i in (0, 1):
                    mk_glob(i, d).wait_send()

    out2 = pl.pallas_call(
        body,
        out_shape=jax.ShapeDtypeStruct((SQ, D), jnp.float32),
        in_specs=[pl.BlockSpec(memory_space=pltpu.VMEM)] * 5,
        out_specs=pl.BlockSpec(memory_space=pltpu.VMEM),
        scratch_shapes=[
            pltpu.VMEM((SQ, D), jnp.bfloat16),
            pltpu.VMEM((HQ, SQ, 1), jnp.float32),
            pltpu.VMEM((HQ, SQ, DH), jnp.float32),
            pltpu.VMEM((HQ, HALO, DH), jnp.bfloat16),
            pltpu.VMEM((HQ, HALO, DH), jnp.bfloat16),
            pltpu.VMEM((HQ, HALO, DH), jnp.bfloat16),
            pltpu.VMEM((HQ, HALO, DH), jnp.bfloat16),
            pltpu.VMEM((HQ, NGLOB, DH), jnp.bfloat16),
            pltpu.VMEM((HQ, NGLOB, DH), jnp.bfloat16),
            pltpu.VMEM((NGLOB, D), jnp.bfloat16),
            pltpu.VMEM((NGLOB, D), jnp.float32),
            pltpu.VMEM((HQ, NGLOB, 1), jnp.float32),
            pltpu.VMEM((3 * NGLOB, D), jnp.float32),
            pltpu.VMEM((3 * HQ, NGLOB, 1), jnp.float32),
            pltpu.SemaphoreType.DMA((4,)),
            pltpu.SemaphoreType.DMA((4,)),
            pltpu.SemaphoreType.DMA((3,)),
            pltpu.SemaphoreType.DMA((3,)),
            pltpu.SemaphoreType.DMA((3,)),
            pltpu.SemaphoreType.DMA((1,)),
            pltpu.SemaphoreType.DMA((1,)),
            pltpu.SemaphoreType.DMA((1,)),
            pltpu.SemaphoreType.DMA((1,)),
            pltpu.SemaphoreType.DMA((1,)),
            pltpu.SemaphoreType.DMA((3,)),
            pltpu.SemaphoreType.DMA((3,)),
        ],
        compiler_params=pltpu.CompilerParams(
            collective_id=0, vmem_limit_bytes=44 * 1024 * 1024),
    )(x2, Wq, Kt, Vt, Wo)
    return out2.reshape(1, SQ, D)
